# baseline (device time: 11024 ns/iter reference)
import jax
import jax.numpy as jnp
from jax import lax
from jax.experimental import pallas as pl
from jax.experimental.pallas import tpu as pltpu

G = 8


def kernel(x):
    m, n = x.shape
    bm = m // G

    def body(x_ref, out_ref, comm_ref, send_sem, recv_sem):
        g = pl.program_id(0)
        my_x = lax.axis_index("x")
        my_y = lax.axis_index("y")
        peer = (1 - my_x, my_y)

        @pl.when(g == 0)
        def _():
            barrier_sem = pltpu.get_barrier_semaphore()
            pl.semaphore_signal(
                barrier_sem, inc=1, device_id=peer,
                device_id_type=pl.DeviceIdType.MESH,
            )
            pl.semaphore_wait(barrier_sem, 1)
            out_ref[:, :] = jnp.max(x_ref[:, :], axis=0, keepdims=True)

        @pl.when(g > 0)
        def _():
            out_ref[:, :] = jnp.maximum(
                out_ref[:, :], jnp.max(x_ref[:, :], axis=0, keepdims=True)
            )

        @pl.when(g == G - 1)
        def _():
            rdma = pltpu.make_async_remote_copy(
                src_ref=out_ref,
                dst_ref=comm_ref,
                send_sem=send_sem,
                recv_sem=recv_sem,
                device_id=peer,
                device_id_type=pl.DeviceIdType.MESH,
            )
            rdma.start()
            rdma.wait()
            out_ref[:, :] = jnp.maximum(out_ref[:, :], comm_ref[:, :])

    return pl.pallas_call(
        body,
        grid=(G,),
        out_shape=jax.ShapeDtypeStruct((1, n), x.dtype),
        in_specs=[pl.BlockSpec((bm, n), lambda g: (g, 0))],
        out_specs=pl.BlockSpec((1, n), lambda g: (0, 0)),
        scratch_shapes=[
            pltpu.VMEM((1, n), x.dtype),
            pltpu.SemaphoreType.DMA,
            pltpu.SemaphoreType.DMA,
        ],
        compiler_params=pltpu.CompilerParams(collective_id=0),
    )(x)
